# baseline (speedup 1.0000x reference)
import functools

import jax
import jax.numpy as jnp
from jax import lax
from jax.experimental import pallas as pl
from jax.experimental.pallas import tpu as pltpu
from jax.experimental.pallas import tpu_sc as plsc

N, D, M, K = 4096, 32, 512, 16
JITTER = 1e-4
RB = 256
NW = 32
RW = N // NW
NT = K * (K + 1) // 2
GROWS = 2 * NT + K
SUB = N // 512
NCH = 8
CH = NT // NCH


def _tables_body(z_ref, lu_raw_ref, mu_ref, ti_ref, lu_ref, murep_ref):
    z = z_ref[...]
    zn = jnp.sum(z * z, axis=1)
    g = jnp.dot(z, z.T, preferred_element_type=jnp.float32)
    d2 = jnp.maximum(zn[:, None] + zn[None, :] - 2.0 * g, 0.0)
    kzz = jnp.exp(-0.5 * d2)
    ri = lax.broadcasted_iota(jnp.int32, (M, M), 0)
    ci = lax.broadcasted_iota(jnp.int32, (M, M), 1)
    eye = (ri == ci).astype(jnp.float32)
    lur = lu_raw_ref[...]
    lu = jnp.where(ri > ci, lur, 0.0) + eye * jnp.exp(lur * eye)
    s = jnp.dot(lu, lu.T, preferred_element_type=jnp.float32)
    h = kzz + (2.0 * JITTER) * eye
    dm = s - kzz - JITTER * eye
    rr = lax.broadcasted_iota(jnp.int32, (M, 2 * M), 0)
    cc = lax.broadcasted_iota(jnp.int32, (M, 2 * M), 1)
    e_even = (cc == 2 * rr).astype(jnp.float32)
    e_odd = (cc == 2 * rr + 1).astype(jnp.float32)
    ti_ref[...] = (
        jnp.dot(h, e_even, preferred_element_type=jnp.float32)
        + jnp.dot(dm, e_odd, preferred_element_type=jnp.float32))
    lu_ref[...] = lu
    murep_ref[...] = jnp.broadcast_to(mu_ref[...], (NW, M))


_tables = pl.pallas_call(
    _tables_body,
    out_shape=(
        jax.ShapeDtypeStruct((M, 2 * M), jnp.float32),
        jax.ShapeDtypeStruct((M, M), jnp.float32),
        jax.ShapeDtypeStruct((NW, M), jnp.float32),
    ),
)


def _topk_body(x_ref, z_ref, idx_ref, kxz_ref):
    x = x_ref[...]
    z = z_ref[...]
    xn = jnp.sum(x * x, axis=1)[:, None]
    zn = jnp.sum(z * z, axis=1)[None, :]
    g = jnp.dot(x, z.T, preferred_element_type=jnp.float32)
    d2 = jnp.maximum(xn + zn - 2.0 * g, 0.0)
    col = lax.broadcasted_iota(jnp.int32, (RB, M), 1)
    work = (lax.bitcast_convert_type(d2, jnp.int32) & jnp.int32(-512)) | col
    big = jnp.int32(0x7FFFFFFF)
    key_cols = []
    for _ in range(K):
        m = jnp.min(work, axis=1, keepdims=True)
        key_cols.append(m)
        work = jnp.where(work == m, big, work)
    keys = jnp.concatenate(key_cols, axis=1)
    idx_blk = keys & jnp.int32(511)
    val_blk = lax.bitcast_convert_type(keys & jnp.int32(-512), jnp.float32)
    idx_ref[...] = idx_blk.T
    kxz_ref[...] = jnp.exp(-0.5 * val_blk).T


@functools.lru_cache(maxsize=None)
def _topk(nc):
    return pl.pallas_call(
        _topk_body,
        grid=(nc // RB,),
        in_specs=[
            pl.BlockSpec((RB, D), lambda i: (i, 0)),
            pl.BlockSpec((M, D), lambda i: (0, 0)),
        ],
        out_specs=(
            pl.BlockSpec((K, RB), lambda i: (0, i)),
            pl.BlockSpec((K, RB), lambda i: (0, i)),
        ),
        out_shape=(
            jax.ShapeDtypeStruct((K, nc), jnp.int32),
            jax.ShapeDtypeStruct((K, nc), jnp.float32),
        ),
    )


@functools.lru_cache(maxsize=None)
def _sc_gather(nc):
    rw = nc // NW

    rdw = max(rw, 128)

    def body(idx_hbm, ti_hbm, mu_hbm, out_hbm, idxv, fidx, gbuf, sem):
        c = lax.axis_index("c")
        s = lax.axis_index("s")
        wid = s * 2 + c
        base = wid * rw
        abase = (base // 128) * 128
        half = base - abase
        pltpu.sync_copy(idx_hbm.at[:, pl.ds(abase, rdw)], idxv)

        for i in range(K):
            @pl.loop(0, rw // 16)
            def _(jc, i=i):
                sl = pl.ds(jc * 16, 16)
                sr = pl.ds(half + jc * 16, 16)
                vi = idxv[i, sr] * (2 * M)
                for j in range(i + 1):
                    f = vi + 2 * idxv[j, sr]
                    fidx[i * (i + 1) // 2 + j, sl] = f
                    fidx[NT + i * (i + 1) // 2 + j, sl] = f + 1
        for i in range(K):
            @pl.loop(0, rw // 16)
            def _(jc, i=i):
                fidx[2 * NT + i, pl.ds(jc * 16, 16)] = (
                    idxv[i, pl.ds(half + jc * 16, 16)] + wid * M)

        def _fire(c):
            @pl.loop(c * CH, (c + 1) * CH)
            def _(p):
                pltpu.async_copy(ti_hbm.at[fidx.at[p]],
                                 gbuf.at[pl.ds(p * rw, rw)], sem)
                pltpu.async_copy(ti_hbm.at[fidx.at[NT + p]],
                                 gbuf.at[pl.ds((NT + p) * rw, rw)], sem)

        def _drain(c):
            for half in (0, NT):
                pltpu.make_async_copy(
                    ti_hbm.at[pl.ds(0, CH * rw)],
                    gbuf.at[pl.ds((half + c * CH) * rw, CH * rw)], sem).wait()

        _fire(0)
        for c in range(1, NCH):
            _fire(c)
            _drain(c - 1)
        _drain(NCH - 1)

        @pl.loop(0, K)
        def _(a):
            pltpu.async_copy(mu_hbm.at[fidx.at[2 * NT + a]],
                             gbuf.at[pl.ds((2 * NT + a) * rw, rw)], sem)
        pltpu.make_async_copy(ti_hbm.at[pl.ds(0, K * rw)],
                              gbuf.at[pl.ds(2 * NT * rw, K * rw)], sem).wait()

        pltpu.sync_copy(gbuf, out_hbm.at[wid])

    return functools.partial(
        pl.kernel,
        out_type=jax.ShapeDtypeStruct((NW, GROWS * (nc // NW)), jnp.float32),
        mesh=plsc.VectorSubcoreMesh(core_axis_name="c", subcore_axis_name="s"),
        scratch_types=[
            pltpu.VMEM((K, rdw), jnp.int32),
            pltpu.VMEM((2 * NT + K, rw), jnp.int32),
            pltpu.VMEM((GROWS * rw,), jnp.float32),
            pltpu.SemaphoreType.DMA,
        ],
    )(body)


def _tri(i, j):
    return i * (i + 1) // 2 + j


def _solve_body(g_ref, b_ref, mean_ref, cov_ref, qf_ref):
    a = {}
    for i in range(K):
        for j in range(i + 1):
            a[(i, j)] = g_ref[_tri(i, j)]
    l = {}
    rinv = [None] * K
    for k in range(K):
        r = lax.rsqrt(a[(k, k)])
        rinv[k] = r
        for i in range(k + 1, K):
            l[(i, k)] = a[(i, k)] * r
        for j in range(k + 1, K):
            for i in range(j, K):
                a[(i, j)] = a[(i, j)] - l[(i, k)] * l[(j, k)]
    y = []
    for i in range(K):
        acc = b_ref[i]
        for k2 in range(i):
            acc = acc - l[(i, k2)] * y[k2]
        y.append(acc * rinv[i])
    w = [None] * K
    for i in reversed(range(K)):
        acc = y[i]
        for k2 in range(i + 1, K):
            acc = acc - l[(k2, i)] * w[k2]
        w[i] = acc * rinv[i]
    q = None
    for i in range(K):
        ti = None
        for j in range(i):
            dij = g_ref[NT + _tri(i, j)]
            ti = dij * w[j] if ti is None else ti + dij * w[j]
        dii = g_ref[NT + _tri(i, i)]
        ti = dii * w[i] if ti is None else 2.0 * ti + dii * w[i]
        q = ti * w[i] if q is None else q + ti * w[i]
    mean = None
    for i in range(K):
        mean = (g_ref[2 * NT + i] * w[i] if mean is None
                else mean + g_ref[2 * NT + i] * w[i])
    cov = 1.0 + q
    mean_ref[...] = mean
    cov_ref[...] = cov
    qf_ref[...] = jnp.sqrt(jnp.maximum(cov, 0.05))


@functools.lru_cache(maxsize=None)
def _solve(nc):
    sub = nc // 512
    return pl.pallas_call(
        _solve_body,
        out_shape=(
            jax.ShapeDtypeStruct((sub, 512), jnp.float32),
            jax.ShapeDtypeStruct((sub, 512), jnp.float32),
            jax.ShapeDtypeStruct((sub, 512), jnp.float32),
        ),
    )


CHUNKS = 4


def kernel(X, Z, Lu_raw, mu):
    ti, lu, murep = _tables(Z, Lu_raw, mu.reshape(1, M))
    murep = murep.reshape(NW * M)
    tif = ti.reshape(2 * M * M)
    nc = N // CHUNKS
    means, covs, qfs = [], [], []
    for c in range(CHUNKS):
        xs = lax.slice_in_dim(X, c * nc, (c + 1) * nc, axis=0)
        idx_t, kxz_t = _topk(nc)(xs, Z)
        g_all = _sc_gather(nc)(idx_t, tif, murep)
        g_all = jnp.swapaxes(
            g_all.reshape(NW, GROWS, nc // NW), 0, 1).reshape(GROWS, nc)
        m8, c8, q8 = _solve(nc)(
            g_all.reshape(GROWS, nc // 512, 512),
            kxz_t.reshape(K, nc // 512, 512))
        means.append(m8.reshape(1, nc))
        covs.append(c8.reshape(1, nc))
        qfs.append(q8.reshape(1, nc))
    mean = jnp.concatenate(means, axis=1)
    cov = jnp.concatenate(covs, axis=1)
    qf = jnp.concatenate(qfs, axis=1)
    return (mean, qf, cov, mu, lu)


# speedup vs baseline: 1.0226x; 1.0226x over previous
import functools

import jax
import jax.numpy as jnp
from jax import lax
from jax.experimental import pallas as pl
from jax.experimental.pallas import tpu as pltpu
from jax.experimental.pallas import tpu_sc as plsc

N, D, M, K = 4096, 32, 512, 16
JITTER = 1e-4
RB = 256
NW = 32
RW = N // NW
NT = K * (K + 1) // 2
GROWS = 2 * NT + K
SUB = N // 512
NCH = 8
CH = NT // NCH


def _tables_body(z_ref, lu_raw_ref, mu_ref, ti_ref, lu_ref, murep_ref):
    z = z_ref[...]
    zn = jnp.sum(z * z, axis=1)
    g = jnp.dot(z, z.T, preferred_element_type=jnp.float32)
    d2 = jnp.maximum(zn[:, None] + zn[None, :] - 2.0 * g, 0.0)
    kzz = jnp.exp(-0.5 * d2)
    ri = lax.broadcasted_iota(jnp.int32, (M, M), 0)
    ci = lax.broadcasted_iota(jnp.int32, (M, M), 1)
    eye = (ri == ci).astype(jnp.float32)
    lur = lu_raw_ref[...]
    lu = jnp.where(ri > ci, lur, 0.0) + eye * jnp.exp(lur * eye)
    s = jnp.dot(lu, lu.T, preferred_element_type=jnp.float32)
    h = kzz + (2.0 * JITTER) * eye
    dm = s - kzz - JITTER * eye
    rr = lax.broadcasted_iota(jnp.int32, (M, 2 * M), 0)
    cc = lax.broadcasted_iota(jnp.int32, (M, 2 * M), 1)
    e_even = (cc == 2 * rr).astype(jnp.float32)
    e_odd = (cc == 2 * rr + 1).astype(jnp.float32)
    ti_ref[...] = (
        jnp.dot(h, e_even, preferred_element_type=jnp.float32)
        + jnp.dot(dm, e_odd, preferred_element_type=jnp.float32))
    lu_ref[...] = lu
    murep_ref[...] = jnp.broadcast_to(mu_ref[...], (NW, M))


_tables = pl.pallas_call(
    _tables_body,
    out_shape=(
        jax.ShapeDtypeStruct((M, 2 * M), jnp.float32),
        jax.ShapeDtypeStruct((M, M), jnp.float32),
        jax.ShapeDtypeStruct((NW, M), jnp.float32),
    ),
)


def _topk_body(x_ref, z_ref, idx_ref, kxz_ref):
    x = x_ref[...]
    z = z_ref[...]
    xn = jnp.sum(x * x, axis=1)[:, None]
    zn = jnp.sum(z * z, axis=1)[None, :]
    g = jnp.dot(x, z.T, preferred_element_type=jnp.float32)
    d2 = jnp.maximum(xn + zn - 2.0 * g, 0.0)
    col = lax.broadcasted_iota(jnp.int32, (RB, M), 1)
    work = (lax.bitcast_convert_type(d2, jnp.int32) & jnp.int32(-512)) | col
    big = jnp.int32(0x7FFFFFFF)
    key_cols = []
    for _ in range(K):
        m = jnp.min(work, axis=1, keepdims=True)
        key_cols.append(m)
        work = jnp.where(work == m, big, work)
    keys = jnp.concatenate(key_cols, axis=1)
    idx_blk = keys & jnp.int32(511)
    val_blk = lax.bitcast_convert_type(keys & jnp.int32(-512), jnp.float32)
    idx_ref[...] = idx_blk.T
    kxz_ref[...] = jnp.exp(-0.5 * val_blk).T


@functools.lru_cache(maxsize=None)
def _topk(nc):
    return pl.pallas_call(
        _topk_body,
        grid=(nc // RB,),
        in_specs=[
            pl.BlockSpec((RB, D), lambda i: (i, 0)),
            pl.BlockSpec((M, D), lambda i: (0, 0)),
        ],
        out_specs=(
            pl.BlockSpec((K, RB), lambda i: (0, i)),
            pl.BlockSpec((K, RB), lambda i: (0, i)),
        ),
        out_shape=(
            jax.ShapeDtypeStruct((K, nc), jnp.int32),
            jax.ShapeDtypeStruct((K, nc), jnp.float32),
        ),
    )


@functools.lru_cache(maxsize=None)
def _sc_gather(nc):
    rw = nc // NW

    rdw = max(rw, 128)

    def body(idx_hbm, ti_hbm, mu_hbm, out_hbm, idxv, fidx, gbuf, sem):
        c = lax.axis_index("c")
        s = lax.axis_index("s")
        wid = s * 2 + c
        base = wid * rw
        abase = (base // 128) * 128
        half = base - abase
        pltpu.sync_copy(idx_hbm.at[:, pl.ds(abase, rdw)], idxv)

        for i in range(K):
            @pl.loop(0, rw // 16)
            def _(jc, i=i):
                sl = pl.ds(jc * 16, 16)
                sr = pl.ds(half + jc * 16, 16)
                vi = idxv[i, sr] * (2 * M)
                for j in range(i + 1):
                    f = vi + 2 * idxv[j, sr]
                    fidx[i * (i + 1) // 2 + j, sl] = f
                    fidx[NT + i * (i + 1) // 2 + j, sl] = f + 1
        for i in range(K):
            @pl.loop(0, rw // 16)
            def _(jc, i=i):
                fidx[2 * NT + i, pl.ds(jc * 16, 16)] = (
                    idxv[i, pl.ds(half + jc * 16, 16)] + wid * M)

        def _fire(c):
            @pl.loop(c * CH, (c + 1) * CH)
            def _(p):
                pltpu.async_copy(ti_hbm.at[fidx.at[p]],
                                 gbuf.at[pl.ds(p * rw, rw)], sem)
                pltpu.async_copy(ti_hbm.at[fidx.at[NT + p]],
                                 gbuf.at[pl.ds((NT + p) * rw, rw)], sem)

        def _drain(c):
            for half in (0, NT):
                pltpu.make_async_copy(
                    ti_hbm.at[pl.ds(0, CH * rw)],
                    gbuf.at[pl.ds((half + c * CH) * rw, CH * rw)], sem).wait()

        _fire(0)
        for c in range(1, NCH):
            _fire(c)
            _drain(c - 1)
        _drain(NCH - 1)

        @pl.loop(0, K)
        def _(a):
            pltpu.async_copy(mu_hbm.at[fidx.at[2 * NT + a]],
                             gbuf.at[pl.ds((2 * NT + a) * rw, rw)], sem)
        pltpu.make_async_copy(ti_hbm.at[pl.ds(0, K * rw)],
                              gbuf.at[pl.ds(2 * NT * rw, K * rw)], sem).wait()

        pltpu.sync_copy(gbuf, out_hbm.at[wid])

    return functools.partial(
        pl.kernel,
        out_type=jax.ShapeDtypeStruct((NW, GROWS * (nc // NW)), jnp.float32),
        mesh=plsc.VectorSubcoreMesh(core_axis_name="c", subcore_axis_name="s"),
        scratch_types=[
            pltpu.VMEM((K, rdw), jnp.int32),
            pltpu.VMEM((2 * NT + K, rw), jnp.int32),
            pltpu.VMEM((GROWS * rw,), jnp.float32),
            pltpu.SemaphoreType.DMA,
        ],
    )(body)


def _tri(i, j):
    return i * (i + 1) // 2 + j


def _solve_body(g_ref, b_ref, mean_ref, cov_ref, qf_ref):
    a = {}
    for i in range(K):
        for j in range(i + 1):
            a[(i, j)] = g_ref[_tri(i, j)]
    l = {}
    rinv = [None] * K
    for k in range(K):
        r = lax.rsqrt(a[(k, k)])
        rinv[k] = r
        for i in range(k + 1, K):
            l[(i, k)] = a[(i, k)] * r
        for j in range(k + 1, K):
            for i in range(j, K):
                a[(i, j)] = a[(i, j)] - l[(i, k)] * l[(j, k)]
    y = []
    for i in range(K):
        acc = b_ref[i]
        for k2 in range(i):
            acc = acc - l[(i, k2)] * y[k2]
        y.append(acc * rinv[i])
    w = [None] * K
    for i in reversed(range(K)):
        acc = y[i]
        for k2 in range(i + 1, K):
            acc = acc - l[(k2, i)] * w[k2]
        w[i] = acc * rinv[i]
    q = None
    for i in range(K):
        ti = None
        for j in range(i):
            dij = g_ref[NT + _tri(i, j)]
            ti = dij * w[j] if ti is None else ti + dij * w[j]
        dii = g_ref[NT + _tri(i, i)]
        ti = dii * w[i] if ti is None else 2.0 * ti + dii * w[i]
        q = ti * w[i] if q is None else q + ti * w[i]
    mean = None
    for i in range(K):
        mean = (g_ref[2 * NT + i] * w[i] if mean is None
                else mean + g_ref[2 * NT + i] * w[i])
    cov = 1.0 + q
    mean_ref[...] = mean
    cov_ref[...] = cov
    qf_ref[...] = jnp.sqrt(jnp.maximum(cov, 0.05))


@functools.lru_cache(maxsize=None)
def _solve(nc):
    sub = nc // 512
    return pl.pallas_call(
        _solve_body,
        out_shape=(
            jax.ShapeDtypeStruct((sub, 512), jnp.float32),
            jax.ShapeDtypeStruct((sub, 512), jnp.float32),
            jax.ShapeDtypeStruct((sub, 512), jnp.float32),
        ),
    )


CHUNKS = 2


def kernel(X, Z, Lu_raw, mu):
    ti, lu, murep = _tables(Z, Lu_raw, mu.reshape(1, M))
    murep = murep.reshape(NW * M)
    tif = ti.reshape(2 * M * M)
    nc = N // CHUNKS
    means, covs, qfs = [], [], []
    for c in range(CHUNKS):
        xs = lax.slice_in_dim(X, c * nc, (c + 1) * nc, axis=0)
        idx_t, kxz_t = _topk(nc)(xs, Z)
        g_all = _sc_gather(nc)(idx_t, tif, murep)
        g_all = jnp.swapaxes(
            g_all.reshape(NW, GROWS, nc // NW), 0, 1).reshape(GROWS, nc)
        m8, c8, q8 = _solve(nc)(
            g_all.reshape(GROWS, nc // 512, 512),
            kxz_t.reshape(K, nc // 512, 512))
        means.append(m8.reshape(1, nc))
        covs.append(c8.reshape(1, nc))
        qfs.append(q8.reshape(1, nc))
    mean = jnp.concatenate(means, axis=1)
    cov = jnp.concatenate(covs, axis=1)
    qf = jnp.concatenate(qfs, axis=1)
    return (mean, qf, cov, mu, lu)
